# Initial kernel scaffold; baseline (speedup 1.0000x reference)
import jax
import jax.numpy as jnp
from jax.experimental import pallas as pl


def kernel(x, W_np, b_np, W1, b1, gamma, beta, W2, b2, Wo, bo):
    raise NotImplementedError("write your pallas kernel here")


# speedup vs baseline: 17.0727x; 17.0727x over previous
import functools

import jax
import jax.numpy as jnp
from jax.experimental import pallas as pl
from jax.experimental.pallas import tpu as pltpu

_NUM_STEPS = 3
_TOPK = 8
_TR = 512


def _f32dot(a, b):
    return jax.lax.dot_general(a, b, (((1,), (0,)), ((), ())),
                               preferred_element_type=jnp.float32)


def _automaton_body(x_ref, Wnp_ref, bnp_ref, W1a_ref, W1b_ref, b1_ref,
                    gamma_ref, beta_ref, W2_ref, b2_ref, Wo_ref, bo_ref,
                    out_ref, state_ref, new_ref):
    n, d = state_ref.shape
    k = _TOPK
    state_ref[...] = x_ref[0]
    Wnp = Wnp_ref[...]
    bnp = bnp_ref[...]
    W1a = W1a_ref[...]
    W1b = W1b_ref[...]
    b1 = b1_ref[...]
    gamma = gamma_ref[...]
    beta = beta_ref[...]
    W2 = W2_ref[...]
    b2 = b2_ref[...]

    for _ in range(_NUM_STEPS):
        st = state_ref[...]
        keys = _f32dot(st, Wnp) + bnp
        nrm = jnp.sqrt(jnp.sum(keys * keys, axis=1, keepdims=True))
        kn = keys / jnp.maximum(nrm, 1e-12)
        for rt in range(n // _TR):
            knt = kn[rt * _TR:(rt + 1) * _TR]
            sim = jax.lax.dot_general(
                knt, kn, (((1,), (1,)), ((), ())),
                preferred_element_type=jnp.float32)
            iota = jax.lax.broadcasted_iota(jnp.int32, (_TR, n), 1)
            s = sim
            onehot = jnp.zeros((_TR, n), jnp.float32)
            for _j in range(k):
                m = jnp.max(s, axis=1, keepdims=True)
                am = jnp.min(jnp.where(s == m, iota, n), axis=1,
                             keepdims=True)
                hit = iota == am
                onehot = onehot + hit.astype(jnp.float32)
                s = jnp.where(hit, -jnp.inf, s)
            nmean = _f32dot(onehot, st) * (1.0 / k)
            agg = _f32dot(nmean, Wnp) + bnp
            stt = st[rt * _TR:(rt + 1) * _TR]
            h = _f32dot(stt, W1a) + _f32dot(agg, W1b) + b1
            mu = jnp.mean(h, axis=1, keepdims=True)
            var = jnp.mean((h - mu) ** 2, axis=1, keepdims=True)
            hn = (h - mu) * jax.lax.rsqrt(var + 1e-5) * gamma + beta
            a = hn * (1.0 / (1.0 + jnp.exp(-hn)))
            new_ref[rt * _TR:(rt + 1) * _TR, :] = stt + _f32dot(a, W2) + b2
        state_ref[...] = new_ref[...]

    meanv = jnp.mean(state_ref[...], axis=0, keepdims=True)
    out_ref[0] = _f32dot(meanv, Wo_ref[...]) + bo_ref[...]


@jax.jit
def kernel(x, W_np, b_np, W1, b1, gamma, beta, W2, b2, Wo, bo):
    B, N, D = x.shape
    H = W_np.shape[1]
    O = Wo.shape[1]
    W1a = W1[:D]
    W1b = W1[D:]
    row = lambda v: v.reshape(1, -1)

    full = lambda shape: pl.BlockSpec(shape, lambda b: (0,) * len(shape))
    out = pl.pallas_call(
        _automaton_body,
        grid=(B,),
        in_specs=[
            pl.BlockSpec((1, N, D), lambda b: (b, 0, 0)),
            full((D, H)), full((1, H)), full((D, H)), full((H, H)),
            full((1, H)), full((1, H)), full((1, H)), full((H, D)),
            full((1, D)), full((D, O)), full((1, O)),
        ],
        out_specs=pl.BlockSpec((1, 1, O), lambda b: (b, 0, 0)),
        out_shape=jax.ShapeDtypeStruct((B, 1, O), jnp.float32),
        scratch_shapes=[
            pltpu.VMEM((N, D), jnp.float32),
            pltpu.VMEM((N, D), jnp.float32),
        ],
    )(x, W_np, row(b_np), W1a, W1b, row(b1), row(gamma), row(beta),
      W2, row(b2), Wo, row(bo))
    return out.reshape(B, O)
